# baseline (speedup 1.0000x reference)
import jax
import jax.numpy as jnp
from jax import lax
from jax.experimental import pallas as pl
from jax.experimental.pallas import tpu as pltpu
from jax.experimental.pallas import tpu_sc as plsc

VOCAB = 1000000
DIM = 64
B = 16384

NC = 2
NS = 16
L = 16
NW = NC * NS
BPW = B // NW
NCOL = (VOCAB + 127) // 128
NQUAD = (NCOL + 3) // 4
QPW = (NQUAD + NW - 1) // NW
PADMINOR = NCOL * 128
NROWS = B + 64
IDXCH = 2048
CAP = B + L


def _log1p_poly(u):
    z = u / (2.0 + u)
    z2 = z * z
    p = 1.0 / 9.0
    p = 1.0 / 7.0 + z2 * p
    p = 1.0 / 5.0 + z2 * p
    p = 1.0 / 3.0 + z2 * p
    p = 1.0 + z2 * p
    return 2.0 * z * p


def _extract_body(c_idx_hbm, t_idx_hbm, w_inT_hbm, w_outT_hbm,
                  rows_c_hbm, rows_t_hbm,
                  idxbuf, clist, tlist, colbuf_c,
                  sbuf_c, sbuf_t, pbuf_c, pbuf_t, estage,
                  sem_cc, sem_ct, sem_sc, sem_st):
    wid = lax.axis_index("s") * NC + lax.axis_index("c")
    p_lo = wid * QPW
    p_hi = jnp.minimum(p_lo + QPW, NQUAD)
    vlo = p_lo * 512
    vhi = jnp.minimum(p_hi * 512, VOCAB)
    lane = lax.iota(jnp.int32, L)
    trash = jnp.int32(B) + 2 * wid

    pltpu.make_async_copy(
        w_inT_hbm.at[pl.ds(0, DIM),
                     pl.ds(jnp.minimum(p_lo * 512, PADMINOR - 512), 512)],
        colbuf_c.at[pl.ds(0, DIM)], sem_cc).start()

    def pack(v, pos):
        rel = jnp.right_shift(v, 9) - p_lo
        return (jnp.left_shift(rel, 23)
                | jnp.left_shift(jnp.bitwise_and(v, 511), 14) | pos)

    def scan_chunk(ch, n_c0):
        pltpu.sync_copy(c_idx_hbm.at[pl.ds(ch * IDXCH, IDXCH)],
                        idxbuf.at[pl.ds(0, IDXCH)])

        def scan_vec(v8, n_c2):
            pos = ch * IDXCH + v8 * L + lane
            cv = idxbuf[pl.ds(v8 * L, L)]
            cm = (cv >= vlo) & (cv < vhi)
            plsc.store_compressed(clist.at[pl.ds(n_c2, L)], pack(cv, pos),
                                  mask=cm)
            return n_c2 + plsc.all_reduce_population_count(cm)[0]

        return lax.fori_loop(0, IDXCH // L, scan_vec, n_c0)

    def scan_t_chunk(ch, n_t0):
        pltpu.sync_copy(t_idx_hbm.at[pl.ds(ch * IDXCH, IDXCH)],
                        idxbuf.at[pl.ds(IDXCH, IDXCH)])

        def scan_vec(v8, n_t2):
            pos = ch * IDXCH + v8 * L + lane
            tv = idxbuf[pl.ds(IDXCH + v8 * L, L)]
            tm = (tv >= vlo) & (tv < vhi)
            plsc.store_compressed(tlist.at[pl.ds(n_t2, L)], pack(tv, pos),
                                  mask=tm)
            return n_t2 + plsc.all_reduce_population_count(tm)[0]

        return lax.fori_loop(0, IDXCH // L, scan_vec, n_t0)

    n_c = lax.fori_loop(0, B // IDXCH, scan_chunk, jnp.int32(0))
    sentinel = jnp.full((L,), -1, jnp.int32)
    clist[pl.ds(n_c, L)] = sentinel

    def pair_dma(pr, par, buf, tbl, sem):
        off = jnp.minimum(pr * 512, PADMINOR - 512)
        return pltpu.make_async_copy(
            tbl.at[pl.ds(0, DIM), pl.ds(off, 512)],
            buf.at[pl.ds(par * DIM, DIM)], sem)

    def table_scan(rel, par, delta, n_s, nlist, plist, colbuf, sbuf, pbuf,
                   rows_hbm, sem):
        def scan_vec(e, n_s2):
            pe = plist[pl.ds(e * L, L)]
            m = jnp.right_shift(pe, 23) == rel
            plsc.store_compressed(estage.at[pl.ds(0, L)], pe, mask=m)
            mm = plsc.all_reduce_population_count(m)[0]

            def elem(j, n_s3):
                slot = jnp.bitwise_and(n_s3, 15)
                bi = jnp.bitwise_and(jnp.right_shift(n_s3, 4), 1)

                @pl.when((slot == 0) & (n_s3 >= 32))
                def _():
                    oldpos = pbuf[pl.ds(bi * L, L)]
                    pltpu.make_async_copy(
                        sbuf.at[pl.ds(bi * 16, 16)],
                        rows_hbm.at[oldpos], sem).wait()

                jv = jnp.full((L,), j, jnp.int32)
                pe_j = plsc.load_gather(estage.at[pl.ds(0, L)], [jv])[0]
                coff = jnp.full(
                    (L,),
                    jnp.bitwise_and(jnp.right_shift(pe_j, 14), 511) + delta)
                pos_j = jnp.bitwise_and(pe_j, 16383)
                for q in range(DIM // L):
                    rows = par * DIM + q * L + lane
                    vq = plsc.load_gather(colbuf, [rows, coff])
                    sbuf[jnp.bitwise_and(n_s3, 31), pl.ds(q * L, L)] = vq
                newpos = jnp.where(lane == slot, pos_j, pbuf[pl.ds(bi * L, L)])
                pbuf[pl.ds(bi * L, L)] = newpos
                n_s3 = n_s3 + 1

                @pl.when(jnp.bitwise_and(n_s3, 15) == 0)
                def _():
                    pltpu.make_async_copy(
                        sbuf.at[pl.ds(bi * 16, 16)],
                        rows_hbm.at[newpos], sem).start()

                return n_s3

            return lax.fori_loop(0, mm, elem, n_s2)

        nvec = (nlist + L - 1) // L
        return lax.fori_loop(0, nvec, scan_vec, n_s)

    def run_pass(tbl, sem_col, nlist, plist, sbuf, pbuf, rows_hbm, sem_s,
                 prologue_started=False):
        if not prologue_started:
            pair_dma(p_lo, jnp.int32(0), colbuf_c, tbl, sem_col).start()

        def sweep(pr, n_s):
            par = lax.rem(pr - p_lo, 2)
            rel = pr - p_lo
            delta = pr * 512 - jnp.minimum(pr * 512, PADMINOR - 512)
            nxt = pr + 1

            @pl.when(nxt < p_hi)
            def _():
                pair_dma(nxt, 1 - par, colbuf_c, tbl, sem_col).start()

            pair_dma(pr, par, colbuf_c, tbl, sem_col).wait()
            return table_scan(rel, par, delta, n_s, nlist, plist, colbuf_c,
                              sbuf, pbuf, rows_hbm, sem_s)

        return lax.fori_loop(p_lo, p_hi, sweep, jnp.int32(0))

    def sweep1(pr, carry):
        n_s, n_t0 = carry
        par = lax.rem(pr - p_lo, 2)
        rel = pr - p_lo
        delta = pr * 512 - jnp.minimum(pr * 512, PADMINOR - 512)
        nxt = pr + 1

        @pl.when(nxt < p_hi)
        def _():
            pair_dma(nxt, 1 - par, colbuf_c, w_inT_hbm, sem_cc).start()

        n_t0 = lax.cond(rel < B // IDXCH,
                        lambda a: scan_t_chunk(rel, a), lambda a: a, n_t0)
        pair_dma(pr, par, colbuf_c, w_inT_hbm, sem_cc).wait()
        n_s = table_scan(rel, par, delta, n_s, n_c, clist, colbuf_c,
                         sbuf_c, pbuf_c, rows_c_hbm, sem_sc)
        return n_s, n_t0

    n_cs, n_t = lax.fori_loop(p_lo, p_hi, sweep1,
                              (jnp.int32(0), jnp.int32(0)))
    tlist[pl.ds(n_t, L)] = sentinel
    n_ts = run_pass(w_outT_hbm, sem_ct, n_t, tlist, sbuf_t, pbuf_t,
                    rows_t_hbm, sem_st)

    def flush_tail(n_s, sbuf, pbuf, rows_hbm, sem, toff):
        rem = jnp.bitwise_and(n_s, 15)
        nf = jnp.right_shift(n_s, 4)
        bi = jnp.bitwise_and(nf, 1)
        bj = jnp.bitwise_and(nf - 1, 1)

        @pl.when(rem != 0)
        def _():
            newpos = jnp.where(lane < rem, pbuf[pl.ds(bi * L, L)],
                               trash + toff)
            pbuf[pl.ds(bi * L, L)] = newpos
            pltpu.make_async_copy(
                sbuf.at[pl.ds(bi * 16, 16)],
                rows_hbm.at[newpos], sem).start()

        @pl.when(nf >= 1)
        def _():
            pltpu.make_async_copy(
                sbuf.at[pl.ds(bj * 16, 16)],
                rows_hbm.at[pbuf[pl.ds(bj * L, L)]], sem).wait()

        @pl.when(rem != 0)
        def _():
            pltpu.make_async_copy(
                sbuf.at[pl.ds(bi * 16, 16)],
                rows_hbm.at[pbuf[pl.ds(bi * L, L)]], sem).wait()

    flush_tail(n_cs, sbuf_c, pbuf_c, rows_c_hbm, sem_sc, 0)
    flush_tail(n_ts, sbuf_t, pbuf_t, rows_t_hbm, sem_st, 1)


def _loss_body(rows_c_hbm, rows_t_hbm, lab_hbm, out_hbm,
               cbuf, tbuf, lab_v, acc_v, sem):
    wid = lax.axis_index("s") * NC + lax.axis_index("c")
    base = wid * BPW
    lane = lax.iota(jnp.int32, L)
    pltpu.sync_copy(lab_hbm.at[pl.ds(base, BPW)], lab_v)

    CHROWS = 64
    NCH = BPW // CHROWS

    def chunk_dma(k, par, src, dstbuf):
        return pltpu.make_async_copy(
            src.at[pl.ds(base + k * CHROWS, CHROWS)],
            dstbuf.at[pl.ds(par * CHROWS, CHROWS)], sem)

    chunk_dma(jnp.int32(0), jnp.int32(0), rows_c_hbm, cbuf).start()
    chunk_dma(jnp.int32(0), jnp.int32(0), rows_t_hbm, tbuf).start()

    def chunk(k, acc):
        par = lax.rem(k, 2)

        @pl.when(k + 1 < NCH)
        def _():
            chunk_dma(k + 1, 1 - par, rows_c_hbm, cbuf).start()
            chunk_dma(k + 1, 1 - par, rows_t_hbm, tbuf).start()

        chunk_dma(k, par, rows_c_hbm, cbuf).wait()
        chunk_dma(k, par, rows_t_hbm, tbuf).wait()

        def group(g, acc2):
            sims = jnp.zeros((L,), jnp.float32)
            for r in range(L):
                row = par * CHROWS + g * L + r
                p = cbuf[row, pl.ds(0, L)] * tbuf[row, pl.ds(0, L)]
                for q in range(1, DIM // L):
                    p = p + cbuf[row, pl.ds(q * L, L)] * tbuf[row, pl.ds(q * L, L)]
                sims = jnp.where(lane == r, jnp.sum(p), sims)
            y = lab_v[pl.ds(k * CHROWS + g * L, L)]
            u = jnp.exp(-jnp.abs(sims))
            loss = jnp.maximum(sims, 0.0) - sims * y + _log1p_poly(u)
            return acc2 + loss

        return lax.fori_loop(0, CHROWS // L, group, acc)

    acc = lax.fori_loop(0, NCH, chunk, jnp.zeros((L,), jnp.float32))
    acc_v[...] = acc
    pltpu.sync_copy(acc_v, out_hbm.at[wid])


_MESH = dict(core_axis_name="c", subcore_axis_name="s")
_PARAMS = dict(needs_layout_passes=False, use_tc_tiling_on_sc=True,
               disable_bounds_checks=True)


@jax.jit
def _sc_call(c_idx, t_idx, lab_f32, w_inT, w_outT):
    rows_c, rows_t = pl.kernel(
        _extract_body,
        out_type=(jax.ShapeDtypeStruct((NROWS, 128), jnp.float32),
                  jax.ShapeDtypeStruct((NROWS, 128), jnp.float32)),
        mesh=plsc.VectorSubcoreMesh(**_MESH),
        compiler_params=pltpu.CompilerParams(**_PARAMS),
        scratch_types=[
            pltpu.VMEM((2 * IDXCH,), jnp.int32),
            pltpu.VMEM((CAP,), jnp.int32),
            pltpu.VMEM((CAP,), jnp.int32),
            pltpu.VMEM((2 * DIM, 512), jnp.float32),
            pltpu.VMEM((32, 128), jnp.float32),
            pltpu.VMEM((32, 128), jnp.float32),
            pltpu.VMEM((2 * L,), jnp.int32),
            pltpu.VMEM((2 * L,), jnp.int32),
            pltpu.VMEM((L,), jnp.int32),
            pltpu.SemaphoreType.DMA,
            pltpu.SemaphoreType.DMA,
            pltpu.SemaphoreType.DMA,
            pltpu.SemaphoreType.DMA,
        ],
    )(c_idx, t_idx, w_inT, w_outT)

    return pl.kernel(
        _loss_body,
        out_type=jax.ShapeDtypeStruct((NW, L), jnp.float32),
        mesh=plsc.VectorSubcoreMesh(**_MESH),
        compiler_params=pltpu.CompilerParams(**_PARAMS),
        scratch_types=[
            pltpu.VMEM((128, 128), jnp.float32),
            pltpu.VMEM((128, 128), jnp.float32),
            pltpu.VMEM((BPW,), jnp.float32),
            pltpu.VMEM((L,), jnp.float32),
            pltpu.SemaphoreType.DMA,
        ],
    )(rows_c, rows_t, lab_f32)


def kernel(center_words, target_words, label, W_in, W_out):
    c_idx = center_words.astype(jnp.int32)
    t_idx = target_words.astype(jnp.int32)
    lab = label.astype(jnp.float32)
    part = _sc_call(c_idx, t_idx, lab, W_in.T, W_out.T)
    return jnp.sum(part) / B


# speedup vs baseline: 1.0249x; 1.0249x over previous
import jax
import jax.numpy as jnp
from jax import lax
from jax.experimental import pallas as pl
from jax.experimental.pallas import tpu as pltpu
from jax.experimental.pallas import tpu_sc as plsc

VOCAB = 1000000
DIM = 64
B = 16384

NC = 2
NS = 16
L = 16
NW = NC * NS
BPW = B // NW
NCOL = (VOCAB + 127) // 128
NQUAD = (NCOL + 3) // 4
QPW = (NQUAD + NW - 1) // NW
PADMINOR = NCOL * 128
NROWS = B + 64
IDXCH = 2048
CAP = B + L


def _log1p_poly(u):
    z = u / (2.0 + u)
    z2 = z * z
    p = 1.0 / 9.0
    p = 1.0 / 7.0 + z2 * p
    p = 1.0 / 5.0 + z2 * p
    p = 1.0 / 3.0 + z2 * p
    p = 1.0 + z2 * p
    return 2.0 * z * p


def _extract_body(c_idx_hbm, t_idx_hbm, w_inT_hbm, w_outT_hbm,
                  rows_c_hbm, rows_t_hbm,
                  idxbuf, clist, tlist, colbuf_c,
                  sbuf_c, sbuf_t, pbuf_c, pbuf_t, estage,
                  sem_cc, sem_ct, sem_sc, sem_st):
    wid = lax.axis_index("s") * NC + lax.axis_index("c")
    p_lo = wid * QPW
    p_hi = jnp.minimum(p_lo + QPW, NQUAD)
    vlo = p_lo * 512
    vhi = jnp.minimum(p_hi * 512, VOCAB)
    lane = lax.iota(jnp.int32, L)
    trash = jnp.int32(B) + 2 * wid

    pltpu.make_async_copy(
        w_inT_hbm.at[pl.ds(0, DIM),
                     pl.ds(jnp.minimum(p_lo * 512, PADMINOR - 512), 512)],
        colbuf_c.at[pl.ds(0, DIM)], sem_cc).start()

    def pack(v, pos):
        rel = jnp.right_shift(v, 9) - p_lo
        return (jnp.left_shift(rel, 23)
                | jnp.left_shift(jnp.bitwise_and(v, 511), 14) | pos)

    def scan_chunk(ch, counts):
        pltpu.sync_copy(c_idx_hbm.at[pl.ds(ch * IDXCH, IDXCH)],
                        idxbuf.at[pl.ds(0, IDXCH)])
        pltpu.sync_copy(t_idx_hbm.at[pl.ds(ch * IDXCH, IDXCH)],
                        idxbuf.at[pl.ds(IDXCH, IDXCH)])

        def scan_vec(v8, counts2):
            n_c2, n_t2 = counts2
            pos = ch * IDXCH + v8 * L + lane
            cv = idxbuf[pl.ds(v8 * L, L)]
            tv = idxbuf[pl.ds(IDXCH + v8 * L, L)]
            cm = (cv >= vlo) & (cv < vhi)
            tm = (tv >= vlo) & (tv < vhi)
            plsc.store_compressed(clist.at[pl.ds(n_c2, L)], pack(cv, pos),
                                  mask=cm)
            plsc.store_compressed(tlist.at[pl.ds(n_t2, L)], pack(tv, pos),
                                  mask=tm)
            n_c2 = n_c2 + plsc.all_reduce_population_count(cm)[0]
            n_t2 = n_t2 + plsc.all_reduce_population_count(tm)[0]
            return n_c2, n_t2

        return lax.fori_loop(0, IDXCH // L, scan_vec, counts)

    n_c, n_t = lax.fori_loop(0, B // IDXCH, scan_chunk,
                             (jnp.int32(0), jnp.int32(0)))
    sentinel = jnp.full((L,), -1, jnp.int32)
    clist[pl.ds(n_c, L)] = sentinel
    tlist[pl.ds(n_t, L)] = sentinel

    def pair_dma(pr, par, buf, tbl, sem):
        off = jnp.minimum(pr * 512, PADMINOR - 512)
        return pltpu.make_async_copy(
            tbl.at[pl.ds(0, DIM), pl.ds(off, 512)],
            buf.at[pl.ds(par * DIM, DIM)], sem)

    def table_scan(rel, par, delta, n_s, nlist, plist, colbuf, sbuf, pbuf,
                   rows_hbm, sem):
        def scan_vec(e, n_s2):
            pe = plist[pl.ds(e * L, L)]
            m = jnp.right_shift(pe, 23) == rel
            plsc.store_compressed(estage.at[pl.ds(0, L)], pe, mask=m)
            mm = plsc.all_reduce_population_count(m)[0]

            def elem(j, n_s3):
                slot = jnp.bitwise_and(n_s3, 15)
                bi = jnp.bitwise_and(jnp.right_shift(n_s3, 4), 1)

                @pl.when((slot == 0) & (n_s3 >= 32))
                def _():
                    oldpos = pbuf[pl.ds(bi * L, L)]
                    pltpu.make_async_copy(
                        sbuf.at[pl.ds(bi * 16, 16)],
                        rows_hbm.at[oldpos], sem).wait()

                jv = jnp.full((L,), j, jnp.int32)
                pe_j = plsc.load_gather(estage.at[pl.ds(0, L)], [jv])[0]
                coff = jnp.full(
                    (L,),
                    jnp.bitwise_and(jnp.right_shift(pe_j, 14), 511) + delta)
                pos_j = jnp.bitwise_and(pe_j, 16383)
                for q in range(DIM // L):
                    rows = par * DIM + q * L + lane
                    vq = plsc.load_gather(colbuf, [rows, coff])
                    sbuf[jnp.bitwise_and(n_s3, 31), pl.ds(q * L, L)] = vq
                newpos = jnp.where(lane == slot, pos_j, pbuf[pl.ds(bi * L, L)])
                pbuf[pl.ds(bi * L, L)] = newpos
                n_s3 = n_s3 + 1

                @pl.when(jnp.bitwise_and(n_s3, 15) == 0)
                def _():
                    pltpu.make_async_copy(
                        sbuf.at[pl.ds(bi * 16, 16)],
                        rows_hbm.at[newpos], sem).start()

                return n_s3

            return lax.fori_loop(0, mm, elem, n_s2)

        nvec = (nlist + L - 1) // L
        return lax.fori_loop(0, nvec, scan_vec, n_s)

    def run_pass(tbl, sem_col, nlist, plist, sbuf, pbuf, rows_hbm, sem_s,
                 prologue_started=False):
        if not prologue_started:
            pair_dma(p_lo, jnp.int32(0), colbuf_c, tbl, sem_col).start()

        def sweep(pr, n_s):
            par = lax.rem(pr - p_lo, 2)
            rel = pr - p_lo
            delta = pr * 512 - jnp.minimum(pr * 512, PADMINOR - 512)
            nxt = pr + 1

            @pl.when(nxt < p_hi)
            def _():
                pair_dma(nxt, 1 - par, colbuf_c, tbl, sem_col).start()

            pair_dma(pr, par, colbuf_c, tbl, sem_col).wait()
            return table_scan(rel, par, delta, n_s, nlist, plist, colbuf_c,
                              sbuf, pbuf, rows_hbm, sem_s)

        return lax.fori_loop(p_lo, p_hi, sweep, jnp.int32(0))

    n_cs = run_pass(w_inT_hbm, sem_cc, n_c, clist, sbuf_c, pbuf_c,
                    rows_c_hbm, sem_sc, prologue_started=True)
    n_ts = run_pass(w_outT_hbm, sem_ct, n_t, tlist, sbuf_t, pbuf_t,
                    rows_t_hbm, sem_st)

    def flush_tail(n_s, sbuf, pbuf, rows_hbm, sem, toff):
        rem = jnp.bitwise_and(n_s, 15)
        nf = jnp.right_shift(n_s, 4)
        bi = jnp.bitwise_and(nf, 1)
        bj = jnp.bitwise_and(nf - 1, 1)

        @pl.when(rem != 0)
        def _():
            newpos = jnp.where(lane < rem, pbuf[pl.ds(bi * L, L)],
                               trash + toff)
            pbuf[pl.ds(bi * L, L)] = newpos
            pltpu.make_async_copy(
                sbuf.at[pl.ds(bi * 16, 16)],
                rows_hbm.at[newpos], sem).start()

        @pl.when(nf >= 1)
        def _():
            pltpu.make_async_copy(
                sbuf.at[pl.ds(bj * 16, 16)],
                rows_hbm.at[pbuf[pl.ds(bj * L, L)]], sem).wait()

        @pl.when(rem != 0)
        def _():
            pltpu.make_async_copy(
                sbuf.at[pl.ds(bi * 16, 16)],
                rows_hbm.at[pbuf[pl.ds(bi * L, L)]], sem).wait()

    flush_tail(n_cs, sbuf_c, pbuf_c, rows_c_hbm, sem_sc, 0)
    flush_tail(n_ts, sbuf_t, pbuf_t, rows_t_hbm, sem_st, 1)


def _loss_body(rows_c_hbm, rows_t_hbm, lab_hbm, out_hbm,
               cbuf, tbuf, lab_v, acc_v, sem):
    wid = lax.axis_index("s") * NC + lax.axis_index("c")
    base = wid * BPW
    lane = lax.iota(jnp.int32, L)
    pltpu.sync_copy(lab_hbm.at[pl.ds(base, BPW)], lab_v)

    CHROWS = 64
    NCH = BPW // CHROWS

    def chunk_dma(k, par, src, dstbuf):
        return pltpu.make_async_copy(
            src.at[pl.ds(base + k * CHROWS, CHROWS)],
            dstbuf.at[pl.ds(par * CHROWS, CHROWS)], sem)

    chunk_dma(jnp.int32(0), jnp.int32(0), rows_c_hbm, cbuf).start()
    chunk_dma(jnp.int32(0), jnp.int32(0), rows_t_hbm, tbuf).start()

    def chunk(k, acc):
        par = lax.rem(k, 2)

        @pl.when(k + 1 < NCH)
        def _():
            chunk_dma(k + 1, 1 - par, rows_c_hbm, cbuf).start()
            chunk_dma(k + 1, 1 - par, rows_t_hbm, tbuf).start()

        chunk_dma(k, par, rows_c_hbm, cbuf).wait()
        chunk_dma(k, par, rows_t_hbm, tbuf).wait()

        def group(g, acc2):
            sims = jnp.zeros((L,), jnp.float32)
            for r in range(L):
                row = par * CHROWS + g * L + r
                p = cbuf[row, pl.ds(0, L)] * tbuf[row, pl.ds(0, L)]
                for q in range(1, DIM // L):
                    p = p + cbuf[row, pl.ds(q * L, L)] * tbuf[row, pl.ds(q * L, L)]
                sims = jnp.where(lane == r, jnp.sum(p), sims)
            y = lab_v[pl.ds(k * CHROWS + g * L, L)]
            u = jnp.exp(-jnp.abs(sims))
            loss = jnp.maximum(sims, 0.0) - sims * y + _log1p_poly(u)
            return acc2 + loss

        return lax.fori_loop(0, CHROWS // L, group, acc)

    acc = lax.fori_loop(0, NCH, chunk, jnp.zeros((L,), jnp.float32))
    acc_v[...] = acc
    pltpu.sync_copy(acc_v, out_hbm.at[wid])


_MESH = dict(core_axis_name="c", subcore_axis_name="s")
_PARAMS = dict(needs_layout_passes=False, use_tc_tiling_on_sc=True,
               disable_bounds_checks=True)


@jax.jit
def _sc_call(c_idx, t_idx, lab_f32, w_inT, w_outT):
    rows_c, rows_t = pl.kernel(
        _extract_body,
        out_type=(jax.ShapeDtypeStruct((NROWS, 128), jnp.float32),
                  jax.ShapeDtypeStruct((NROWS, 128), jnp.float32)),
        mesh=plsc.VectorSubcoreMesh(**_MESH),
        compiler_params=pltpu.CompilerParams(**_PARAMS),
        scratch_types=[
            pltpu.VMEM((2 * IDXCH,), jnp.int32),
            pltpu.VMEM((CAP,), jnp.int32),
            pltpu.VMEM((CAP,), jnp.int32),
            pltpu.VMEM((2 * DIM, 512), jnp.float32),
            pltpu.VMEM((32, 128), jnp.float32),
            pltpu.VMEM((32, 128), jnp.float32),
            pltpu.VMEM((2 * L,), jnp.int32),
            pltpu.VMEM((2 * L,), jnp.int32),
            pltpu.VMEM((L,), jnp.int32),
            pltpu.SemaphoreType.DMA,
            pltpu.SemaphoreType.DMA,
            pltpu.SemaphoreType.DMA,
            pltpu.SemaphoreType.DMA,
        ],
    )(c_idx, t_idx, w_inT, w_outT)

    return pl.kernel(
        _loss_body,
        out_type=jax.ShapeDtypeStruct((NW, L), jnp.float32),
        mesh=plsc.VectorSubcoreMesh(**_MESH),
        compiler_params=pltpu.CompilerParams(**_PARAMS),
        scratch_types=[
            pltpu.VMEM((128, 128), jnp.float32),
            pltpu.VMEM((128, 128), jnp.float32),
            pltpu.VMEM((BPW,), jnp.float32),
            pltpu.VMEM((L,), jnp.float32),
            pltpu.SemaphoreType.DMA,
        ],
    )(rows_c, rows_t, lab_f32)


def kernel(center_words, target_words, label, W_in, W_out):
    c_idx = center_words.astype(jnp.int32)
    t_idx = target_words.astype(jnp.int32)
    lab = label.astype(jnp.float32)
    part = _sc_call(c_idx, t_idx, lab, W_in.T, W_out.T)
    return jnp.sum(part) / B
